# baseline (speedup 1.0000x reference)
import functools

import jax
import jax.numpy as jnp
from jax import lax
from jax.experimental import pallas as pl
from jax.experimental.pallas import tpu as pltpu
from jax.experimental.pallas import tpu_sc as plsc

BN = 512
HT = 1024
W = 16
PW = 128


def _routing_body(nslots, n_e, bn, nsteps, nh, x_ref, gwt_ref, pos_ref, pb_ref,
                  meta_ref, oh_ref, ex_ref):
    n = x_ref.shape[0]
    scores = jnp.dot(x_ref[...], gwt_ref[...],
                     preferred_element_type=jnp.float32)
    cols = lax.broadcasted_iota(jnp.int32, (n, n_e), 1)
    s0 = jnp.max(scores, axis=1, keepdims=True)
    e0 = jnp.min(jnp.where(scores == s0, cols, n_e), axis=1, keepdims=True)
    masked = jnp.where(cols == e0, -jnp.inf, scores)
    s1 = jnp.max(masked, axis=1, keepdims=True)
    e1 = jnp.min(jnp.where(masked == s1, cols, n_e), axis=1, keepdims=True)
    t = jnp.exp(s1 - s0)
    denom = 1.0 + t
    p0 = 1.0 / denom
    p1 = t / denom

    oh_ref[:n, :] = (cols == e0).astype(jnp.float32)
    oh_ref[n:, :] = (cols == e1).astype(jnp.float32)

    ch = 512
    tri = (lax.broadcasted_iota(jnp.int32, (ch, ch), 0)
           > lax.broadcasted_iota(jnp.int32, (ch, ch), 1)).astype(jnp.float32)

    def chunk(i, carry):
        blk = oh_ref[pl.ds(i * ch, ch), :]
        ex_ref[pl.ds(i * ch, ch), :] = (
            jnp.dot(tri, blk, preferred_element_type=jnp.float32) + carry)
        return carry + jnp.sum(blk, axis=0, keepdims=True)

    counts = lax.fori_loop(0, (2 * n) // ch, chunk,
                           jnp.zeros((1, n_e), jnp.float32))
    ranks = jnp.sum(ex_ref[...] * oh_ref[...], axis=1, keepdims=True)

    ci = counts.astype(jnp.int32)
    nb = (ci + (bn - 1)) // bn
    pc = nb * bn
    upper = (lax.broadcasted_iota(jnp.int32, (n_e, n_e), 0)
             < lax.broadcasted_iota(jnp.int32, (n_e, n_e), 1)).astype(jnp.float32)
    aoff = jnp.dot(pc.astype(jnp.float32), upper,
                   preferred_element_type=jnp.float32)
    asel = jnp.sum(oh_ref[...] * aoff, axis=1, keepdims=True)
    pos_ref[...] = (asel + ranks).astype(jnp.int32)
    pb_ref[...] = jnp.broadcast_to(jnp.concatenate([p0, p1], axis=0),
                                   (nslots, pb_ref.shape[1]))

    steps_e = nb * nh
    cums = jnp.dot(steps_e.astype(jnp.float32), upper,
                   preferred_element_type=jnp.float32).astype(jnp.int32)
    cums_inc = cums + steps_e
    nreal = jnp.sum(steps_e, axis=1, keepdims=True)
    sidx = lax.broadcasted_iota(jnp.int32, (1, nsteps), 1)
    ge = (sidx >= cums_inc.reshape(n_e, 1)).astype(jnp.int32)
    e_of_s = jnp.sum(ge, axis=0, keepdims=True)
    e_clamped = jnp.minimum(e_of_s, n_e - 1)
    sel = (e_clamped == lax.broadcasted_iota(jnp.int32, (n_e, nsteps), 0)
           ).astype(jnp.int32)
    base_sel = jnp.sum(sel * cums.reshape(n_e, 1), axis=0, keepdims=True)
    nb_sel = jnp.sum(sel * nb.reshape(n_e, 1), axis=0, keepdims=True)
    ab_sel = jnp.sum(sel * (aoff.astype(jnp.int32) // bn).reshape(n_e, 1),
                     axis=0, keepdims=True)
    r = sidx - base_sel
    nb_safe = jnp.maximum(nb_sel, 1)
    h = (r.astype(jnp.float32) / nb_safe.astype(jnp.float32)).astype(jnp.int32)
    j = r - h * nb_sel
    valid = (sidx < nreal).astype(jnp.int32)
    dump = (nslots + n_e * bn) // bn - 1
    rb = jnp.where(valid == 1, ab_sel + j, dump)
    h = jnp.where(valid == 1, h, nh - 1)
    j = jnp.where(valid == 1, j, 0)
    islast = jnp.where(valid == 1, (h == nh - 1).astype(jnp.int32), 0)
    meta_ref[0:1, :] = e_clamped
    meta_ref[1:2, :] = h
    meta_ref[2:3, :] = rb
    meta_ref[3:4, :] = j
    meta_ref[4:5, :] = islast
    meta_ref[5:6, :] = valid
    meta_ref[6:7, :] = jnp.where(islast == 1, rb, dump)
    meta_ref[7:8, :] = jnp.zeros((1, nsteps), jnp.int32)


def _ffn_body(bn, meta_ref, xs_ref, w1_ref, w2_ref, w3_ref, ps_ref, ys_ref,
              acc_ref):
    s = pl.program_id(0)
    h = meta_ref[1, s]
    j = meta_ref[3, s]
    islast = meta_ref[4, s]
    valid = meta_ref[5, s]
    base = j * bn

    @pl.when(valid == 1)
    def _():
        xb = xs_ref[...]
        w1b = w1_ref[0].astype(jnp.bfloat16)
        w2b = w2_ref[0].astype(jnp.bfloat16)
        w3b = w3_ref[0].astype(jnp.bfloat16)
        dn = (((1,), (1,)), ((), ()))
        a = lax.dot_general(xb, w1b, dn, preferred_element_type=jnp.float32)
        b = lax.dot_general(xb, w2b, dn, preferred_element_type=jnp.float32)
        hid = (a * (1.0 / (1.0 + jnp.exp(-a)))) * b
        part = lax.dot_general(hid.astype(jnp.bfloat16), w3b, dn,
                               preferred_element_type=jnp.float32)
        prev = acc_ref[pl.ds(base, bn), :]
        acc = jnp.where(h == 0, part, prev + part)
        acc_ref[pl.ds(base, bn), :] = acc

        @pl.when(islast == 1)
        def _():
            ys_ref[...] = acc * ps_ref[:, 0:1]


def _sc_scatter(x_flat, posf, pb, p_rows):
    n, d = x_flat.shape
    mesh = plsc.VectorSubcoreMesh(core_axis_name="c", subcore_axis_name="s")
    nworkers = 32
    tpw = n // nworkers
    nchunks = tpw // W

    @functools.partial(
        pl.kernel,
        out_type=[
            jax.ShapeDtypeStruct((p_rows, d), jnp.int32),
            jax.ShapeDtypeStruct((p_rows, PW), jnp.float32),
        ],
        mesh=mesh,
        scratch_types=[
            pltpu.VMEM((W, d), jnp.int32),
            pltpu.VMEM((W, d), jnp.int32),
            pltpu.VMEM((W, PW), jnp.float32),
            pltpu.VMEM((W, PW), jnp.float32),
            pltpu.VMEM((W, PW), jnp.float32),
            pltpu.VMEM((W, PW), jnp.float32),
            pltpu.VMEM((W,), jnp.int32),
            pltpu.VMEM((W,), jnp.int32),
            pltpu.VMEM((W,), jnp.int32),
            pltpu.VMEM((W,), jnp.int32),
            pltpu.SemaphoreType.DMA,
            pltpu.SemaphoreType.DMA,
            pltpu.SemaphoreType.DMA,
            pltpu.SemaphoreType.DMA,
        ],
    )
    def k(x_hbm, pos_hbm, pb_hbm, xs_hbm, ps_hbm, xva, xvb,
          pv0a, pv0b, pv1a, pv1b, i0a, i0b, i1a, i1b, sla, slb, ssa, ssb):
        wid = lax.axis_index("s") * 2 + lax.axis_index("c")
        xv = (xva, xvb)
        pv0 = (pv0a, pv0b)
        pv1 = (pv1a, pv1b)
        i0 = (i0a, i0b)
        i1 = (i1a, i1b)
        sl = (sla, slb)
        ss = (ssa, ssb)

        def start(cc):
            bsl = cc % 2
            base = wid * tpw + cc * W
            pltpu.sync_copy(pos_hbm.at[pl.ds(base, W)], i0[bsl])
            pltpu.sync_copy(pos_hbm.at[pl.ds(n + base, W)], i1[bsl])
            pltpu.sync_copy(pb_hbm.at[pl.ds(base, W)], pv0[bsl])
            pltpu.sync_copy(pb_hbm.at[pl.ds(n + base, W)], pv1[bsl])
            return pltpu.async_copy(x_hbm.at[pl.ds(base, W)], xv[bsl], sl[bsl])

        loads = start(0)
        stores = [None, None]
        for cc in range(nchunks):
            bsl = cc % 2
            loads.wait()
            cps = (
                pltpu.async_copy(xv[bsl], xs_hbm.at[i0[bsl]], ss[bsl]),
                pltpu.async_copy(xv[bsl], xs_hbm.at[i1[bsl]], ss[bsl]),
                pltpu.async_copy(pv0[bsl], ps_hbm.at[i0[bsl]], ss[bsl]),
                pltpu.async_copy(pv1[bsl], ps_hbm.at[i1[bsl]], ss[bsl]),
            )
            stores[bsl] = cps
            if cc + 1 < nchunks:
                nxt = (cc + 1) % 2
                if stores[nxt] is not None:
                    for cp in stores[nxt]:
                        cp.wait()
                    stores[nxt] = None
                loads = start(cc + 1)
        for group in stores:
            if group is not None:
                for cp in group:
                    cp.wait()

    return k(x_flat, posf, pb)


def _sc_combine(ys, posf, n, d):
    mesh = plsc.VectorSubcoreMesh(core_axis_name="c", subcore_axis_name="s")
    nworkers = 32
    tpw = n // nworkers
    nchunks = tpw // W

    @functools.partial(
        pl.kernel,
        out_type=jax.ShapeDtypeStruct((n, d), jnp.float32),
        mesh=mesh,
        scratch_types=[
            pltpu.VMEM((W, d), jnp.float32),
            pltpu.VMEM((W, d), jnp.float32),
            pltpu.VMEM((W, d), jnp.float32),
            pltpu.VMEM((W, d), jnp.float32),
            pltpu.VMEM((tpw,), jnp.int32),
            pltpu.VMEM((tpw,), jnp.int32),
            pltpu.SemaphoreType.DMA,
            pltpu.SemaphoreType.DMA,
            pltpu.SemaphoreType.DMA,
            pltpu.SemaphoreType.DMA,
        ],
    )
    def k(ys_hbm, pos_hbm, out_hbm, g0a, g0b, g1a, g1b,
          i0all, i1all, sga, sgb, soa, sob):
        wid = lax.axis_index("s") * 2 + lax.axis_index("c")
        base0 = wid * tpw
        pltpu.sync_copy(pos_hbm.at[pl.ds(base0, tpw)], i0all)
        pltpu.sync_copy(pos_hbm.at[pl.ds(n + base0, tpw)], i1all)
        g0 = (g0a, g0b)
        g1 = (g1a, g1b)
        sg = (sga, sgb)
        so = (soa, sob)

        def start(cc):
            bsl = cc % 2
            c0 = pltpu.async_copy(
                ys_hbm.at[i0all.at[pl.ds(cc * W, W)]], g0[bsl], sg[bsl])
            c1 = pltpu.async_copy(
                ys_hbm.at[i1all.at[pl.ds(cc * W, W)]], g1[bsl], sg[bsl])
            return (c0, c1)

        loads = start(0)
        stores = [None, None]
        for cc in range(nchunks):
            bsl = cc % 2
            for cp in loads:
                cp.wait()
            if cc + 1 < nchunks:
                nxt = (cc + 1) % 2
                if stores[nxt] is not None:
                    stores[nxt].wait()
                    stores[nxt] = None
                loads = start(cc + 1)
            for rr in range(W):

                @pl.loop(0, d // W)
                def _(c):
                    csl = pl.ds(c * W, W)
                    g0[bsl][rr, csl] = g0[bsl][rr, csl] + g1[bsl][rr, csl]

            stores[bsl] = pltpu.async_copy(
                g0[bsl], out_hbm.at[pl.ds(base0 + cc * W, W)], so[bsl])
        for st in stores:
            if st is not None:
                st.wait()

    return k(ys, posf)


def kernel(x, gate_w, w1, w2, w3):
    b, s, d = x.shape
    n_e, _ = gate_w.shape
    hdim = w1.shape[1]
    n = b * s
    nslots = 2 * n
    nh = hdim // HT
    p_rows = nslots + n_e * BN
    nsteps = (p_rows // BN) * nh

    x_flat = x.reshape(n, d)
    gwt = gate_w.T

    routing = pl.pallas_call(
        functools.partial(_routing_body, nslots, n_e, BN, nsteps, nh),
        out_shape=[
            jax.ShapeDtypeStruct((nslots, 1), jnp.int32),
            jax.ShapeDtypeStruct((nslots, PW), jnp.float32),
            jax.ShapeDtypeStruct((8, nsteps), jnp.int32),
        ],
        scratch_shapes=[
            pltpu.VMEM((nslots, n_e), jnp.float32),
            pltpu.VMEM((nslots, n_e), jnp.float32),
        ],
    )
    pos2, pb, meta = routing(x_flat, gwt)
    posf = pos2.reshape(nslots)

    x_bf = x_flat.astype(jnp.bfloat16)
    x_i32 = lax.bitcast_convert_type(x_bf.reshape(n, d // 2, 2), jnp.int32)
    xs_i32, ps = _sc_scatter(x_i32, posf, pb, p_rows)
    xs = lax.bitcast_convert_type(xs_i32, jnp.bfloat16).reshape(p_rows, d)

    grid_spec = pltpu.PrefetchScalarGridSpec(
        num_scalar_prefetch=1,
        grid=(nsteps,),
        in_specs=[
            pl.BlockSpec((BN, d), lambda s_, m: (m[2, s_], 0)),
            pl.BlockSpec((1, HT, d), lambda s_, m: (m[0, s_], m[1, s_], 0)),
            pl.BlockSpec((1, HT, d), lambda s_, m: (m[0, s_], m[1, s_], 0)),
            pl.BlockSpec((1, d, HT), lambda s_, m: (m[0, s_], 0, m[1, s_])),
            pl.BlockSpec((BN, PW), lambda s_, m: (m[2, s_], 0)),
        ],
        out_specs=pl.BlockSpec((BN, d), lambda s_, m: (m[6, s_], 0)),
        scratch_shapes=[pltpu.VMEM((n, d), jnp.float32)],
    )
    ys = pl.pallas_call(
        functools.partial(_ffn_body, BN),
        grid_spec=grid_spec,
        out_shape=jax.ShapeDtypeStruct((p_rows, d), jnp.float32),
    )(meta, xs, w1, w2, w3, ps)

    out = _sc_combine(ys, posf, n, d)
    return out.reshape(b, s, d)


# speedup vs baseline: 1.6814x; 1.6814x over previous
import functools

import jax
import jax.numpy as jnp
from jax import lax
from jax.experimental import pallas as pl
from jax.experimental.pallas import tpu as pltpu
from jax.experimental.pallas import tpu_sc as plsc

BN = 512
HT = 1024
W = 16
PW = 128


def _routing_body(nslots, n_e, bn, nsteps, nh, x_ref, gwt_ref, pos_ref, pb_ref,
                  meta_ref, oh_ref, ex_ref):
    n = x_ref.shape[0]
    scores = jnp.dot(x_ref[...], gwt_ref[...],
                     preferred_element_type=jnp.float32)
    cols = lax.broadcasted_iota(jnp.int32, (n, n_e), 1)
    s0 = jnp.max(scores, axis=1, keepdims=True)
    e0 = jnp.min(jnp.where(scores == s0, cols, n_e), axis=1, keepdims=True)
    masked = jnp.where(cols == e0, -jnp.inf, scores)
    s1 = jnp.max(masked, axis=1, keepdims=True)
    e1 = jnp.min(jnp.where(masked == s1, cols, n_e), axis=1, keepdims=True)
    t = jnp.exp(s1 - s0)
    denom = 1.0 + t
    p0 = 1.0 / denom
    p1 = t / denom

    oh_ref[:n, :] = (cols == e0).astype(jnp.float32)
    oh_ref[n:, :] = (cols == e1).astype(jnp.float32)

    ch = 512
    tri = (lax.broadcasted_iota(jnp.int32, (ch, ch), 0)
           > lax.broadcasted_iota(jnp.int32, (ch, ch), 1)).astype(jnp.float32)

    def chunk(i, carry):
        blk = oh_ref[pl.ds(i * ch, ch), :]
        ex_ref[pl.ds(i * ch, ch), :] = (
            jnp.dot(tri, blk, preferred_element_type=jnp.float32) + carry)
        return carry + jnp.sum(blk, axis=0, keepdims=True)

    counts = lax.fori_loop(0, (2 * n) // ch, chunk,
                           jnp.zeros((1, n_e), jnp.float32))
    ranks = jnp.sum(ex_ref[...] * oh_ref[...], axis=1, keepdims=True)

    ci = counts.astype(jnp.int32)
    nb = (ci + (bn - 1)) // bn
    pc = nb * bn
    upper = (lax.broadcasted_iota(jnp.int32, (n_e, n_e), 0)
             < lax.broadcasted_iota(jnp.int32, (n_e, n_e), 1)).astype(jnp.float32)
    aoff = jnp.dot(pc.astype(jnp.float32), upper,
                   preferred_element_type=jnp.float32)
    asel = jnp.sum(oh_ref[...] * aoff, axis=1, keepdims=True)
    pos_ref[...] = (asel + ranks).astype(jnp.int32)
    pb_ref[...] = jnp.broadcast_to(jnp.concatenate([p0, p1], axis=0),
                                   (nslots, pb_ref.shape[1]))

    steps_e = nb * nh
    cums = jnp.dot(steps_e.astype(jnp.float32), upper,
                   preferred_element_type=jnp.float32).astype(jnp.int32)
    cums_inc = cums + steps_e
    nreal = jnp.sum(steps_e, axis=1, keepdims=True)
    sidx = lax.broadcasted_iota(jnp.int32, (1, nsteps), 1)
    ge = (sidx >= cums_inc.reshape(n_e, 1)).astype(jnp.int32)
    e_of_s = jnp.sum(ge, axis=0, keepdims=True)
    e_clamped = jnp.minimum(e_of_s, n_e - 1)
    sel = (e_clamped == lax.broadcasted_iota(jnp.int32, (n_e, nsteps), 0)
           ).astype(jnp.int32)
    base_sel = jnp.sum(sel * cums.reshape(n_e, 1), axis=0, keepdims=True)
    nb_sel = jnp.sum(sel * nb.reshape(n_e, 1), axis=0, keepdims=True)
    ab_sel = jnp.sum(sel * (aoff.astype(jnp.int32) // bn).reshape(n_e, 1),
                     axis=0, keepdims=True)
    r = sidx - base_sel
    nb_safe = jnp.maximum(nb_sel, 1)
    h = (r.astype(jnp.float32) / nb_safe.astype(jnp.float32)).astype(jnp.int32)
    j = r - h * nb_sel
    valid = (sidx < nreal).astype(jnp.int32)
    dump = (nslots + n_e * bn) // bn - 1
    rb = jnp.where(valid == 1, ab_sel + j, dump)
    h = jnp.where(valid == 1, h, nh - 1)
    j = jnp.where(valid == 1, j, 0)
    islast = jnp.where(valid == 1, (h == nh - 1).astype(jnp.int32), 0)
    meta_ref[0:1, :] = e_clamped
    meta_ref[1:2, :] = h
    meta_ref[2:3, :] = rb
    meta_ref[3:4, :] = j
    meta_ref[4:5, :] = islast
    meta_ref[5:6, :] = valid
    meta_ref[6:7, :] = jnp.where(islast == 1, rb, dump)
    meta_ref[7:8, :] = jnp.zeros((1, nsteps), jnp.int32)


def _ffn_body(bn, meta_ref, xs_ref, w1_ref, w2_ref, w3_ref, ps_ref, ys_ref,
              acc_ref):
    s = pl.program_id(0)
    h = meta_ref[1, s]
    j = meta_ref[3, s]
    islast = meta_ref[4, s]
    valid = meta_ref[5, s]
    base = j * bn

    @pl.when(valid == 1)
    def _():
        xb = xs_ref[...].astype(jnp.bfloat16)
        w1b = w1_ref[0].astype(jnp.bfloat16)
        w2b = w2_ref[0].astype(jnp.bfloat16)
        w3b = w3_ref[0].astype(jnp.bfloat16)
        dn = (((1,), (1,)), ((), ()))
        a = lax.dot_general(xb, w1b, dn, preferred_element_type=jnp.float32)
        b = lax.dot_general(xb, w2b, dn, preferred_element_type=jnp.float32)
        hid = (a * (1.0 / (1.0 + jnp.exp(-a)))) * b
        part = lax.dot_general(hid.astype(jnp.bfloat16), w3b, dn,
                               preferred_element_type=jnp.float32)
        prev = acc_ref[pl.ds(base, bn), :]
        acc = jnp.where(h == 0, part, prev + part)
        acc_ref[pl.ds(base, bn), :] = acc

        @pl.when(islast == 1)
        def _():
            ys_ref[...] = acc * ps_ref[:, 0:1]


def _sc_scatter(x_flat, posf, pb, p_rows):
    n, d = x_flat.shape
    mesh = plsc.VectorSubcoreMesh(core_axis_name="c", subcore_axis_name="s")
    nworkers = 32
    tpw = n // nworkers
    nchunks = tpw // W

    @functools.partial(
        pl.kernel,
        out_type=[
            jax.ShapeDtypeStruct((p_rows, d), jnp.float32),
            jax.ShapeDtypeStruct((p_rows, PW), jnp.float32),
        ],
        mesh=mesh,
        scratch_types=[
            pltpu.VMEM((W, d), jnp.float32),
            pltpu.VMEM((W, d), jnp.float32),
            pltpu.VMEM((W, PW), jnp.float32),
            pltpu.VMEM((W, PW), jnp.float32),
            pltpu.VMEM((W, PW), jnp.float32),
            pltpu.VMEM((W, PW), jnp.float32),
            pltpu.VMEM((W,), jnp.int32),
            pltpu.VMEM((W,), jnp.int32),
            pltpu.VMEM((W,), jnp.int32),
            pltpu.VMEM((W,), jnp.int32),
            pltpu.SemaphoreType.DMA,
            pltpu.SemaphoreType.DMA,
            pltpu.SemaphoreType.DMA,
            pltpu.SemaphoreType.DMA,
        ],
    )
    def k(x_hbm, pos_hbm, pb_hbm, xs_hbm, ps_hbm, xva, xvb,
          pv0a, pv0b, pv1a, pv1b, i0a, i0b, i1a, i1b, sla, slb, ssa, ssb):
        wid = lax.axis_index("s") * 2 + lax.axis_index("c")
        xv = (xva, xvb)
        pv0 = (pv0a, pv0b)
        pv1 = (pv1a, pv1b)
        i0 = (i0a, i0b)
        i1 = (i1a, i1b)
        sl = (sla, slb)
        ss = (ssa, ssb)

        def start(cc):
            bsl = cc % 2
            base = wid * tpw + cc * W
            pltpu.sync_copy(pos_hbm.at[pl.ds(base, W)], i0[bsl])
            pltpu.sync_copy(pos_hbm.at[pl.ds(n + base, W)], i1[bsl])
            pltpu.sync_copy(pb_hbm.at[pl.ds(base, W)], pv0[bsl])
            pltpu.sync_copy(pb_hbm.at[pl.ds(n + base, W)], pv1[bsl])
            return pltpu.async_copy(x_hbm.at[pl.ds(base, W)], xv[bsl], sl[bsl])

        loads = start(0)
        stores = [None, None]
        for cc in range(nchunks):
            bsl = cc % 2
            loads.wait()
            cps = (
                pltpu.async_copy(xv[bsl], xs_hbm.at[i0[bsl]], ss[bsl]),
                pltpu.async_copy(xv[bsl], xs_hbm.at[i1[bsl]], ss[bsl]),
                pltpu.async_copy(pv0[bsl], ps_hbm.at[i0[bsl]], ss[bsl]),
                pltpu.async_copy(pv1[bsl], ps_hbm.at[i1[bsl]], ss[bsl]),
            )
            stores[bsl] = cps
            if cc + 1 < nchunks:
                nxt = (cc + 1) % 2
                if stores[nxt] is not None:
                    for cp in stores[nxt]:
                        cp.wait()
                    stores[nxt] = None
                loads = start(cc + 1)
        for group in stores:
            if group is not None:
                for cp in group:
                    cp.wait()

    return k(x_flat, posf, pb)


def _sc_combine(ys, posf, n, d):
    mesh = plsc.VectorSubcoreMesh(core_axis_name="c", subcore_axis_name="s")
    nworkers = 32
    tpw = n // nworkers
    nchunks = tpw // W

    @functools.partial(
        pl.kernel,
        out_type=jax.ShapeDtypeStruct((n, d), jnp.float32),
        mesh=mesh,
        scratch_types=[
            pltpu.VMEM((W, d), jnp.float32),
            pltpu.VMEM((W, d), jnp.float32),
            pltpu.VMEM((W, d), jnp.float32),
            pltpu.VMEM((W, d), jnp.float32),
            pltpu.VMEM((tpw,), jnp.int32),
            pltpu.VMEM((tpw,), jnp.int32),
            pltpu.SemaphoreType.DMA,
            pltpu.SemaphoreType.DMA,
            pltpu.SemaphoreType.DMA,
            pltpu.SemaphoreType.DMA,
        ],
    )
    def k(ys_hbm, pos_hbm, out_hbm, g0a, g0b, g1a, g1b,
          i0all, i1all, sga, sgb, soa, sob):
        wid = lax.axis_index("s") * 2 + lax.axis_index("c")
        base0 = wid * tpw
        pltpu.sync_copy(pos_hbm.at[pl.ds(base0, tpw)], i0all)
        pltpu.sync_copy(pos_hbm.at[pl.ds(n + base0, tpw)], i1all)
        g0 = (g0a, g0b)
        g1 = (g1a, g1b)
        sg = (sga, sgb)
        so = (soa, sob)

        def start(cc):
            bsl = cc % 2
            c0 = pltpu.async_copy(
                ys_hbm.at[i0all.at[pl.ds(cc * W, W)]], g0[bsl], sg[bsl])
            c1 = pltpu.async_copy(
                ys_hbm.at[i1all.at[pl.ds(cc * W, W)]], g1[bsl], sg[bsl])
            return (c0, c1)

        loads = start(0)
        stores = [None, None]
        for cc in range(nchunks):
            bsl = cc % 2
            for cp in loads:
                cp.wait()
            if cc + 1 < nchunks:
                nxt = (cc + 1) % 2
                if stores[nxt] is not None:
                    stores[nxt].wait()
                    stores[nxt] = None
                loads = start(cc + 1)
            for rr in range(W):

                @pl.loop(0, d // W)
                def _(c):
                    csl = pl.ds(c * W, W)
                    g0[bsl][rr, csl] = g0[bsl][rr, csl] + g1[bsl][rr, csl]

            stores[bsl] = pltpu.async_copy(
                g0[bsl], out_hbm.at[pl.ds(base0 + cc * W, W)], so[bsl])
        for st in stores:
            if st is not None:
                st.wait()

    return k(ys, posf)


def kernel(x, gate_w, w1, w2, w3):
    b, s, d = x.shape
    n_e, _ = gate_w.shape
    hdim = w1.shape[1]
    n = b * s
    nslots = 2 * n
    nh = hdim // HT
    p_rows = nslots + n_e * BN
    nsteps = (p_rows // BN) * nh

    x_flat = x.reshape(n, d)
    gwt = gate_w.T

    routing = pl.pallas_call(
        functools.partial(_routing_body, nslots, n_e, BN, nsteps, nh),
        out_shape=[
            jax.ShapeDtypeStruct((nslots, 1), jnp.int32),
            jax.ShapeDtypeStruct((nslots, PW), jnp.float32),
            jax.ShapeDtypeStruct((8, nsteps), jnp.int32),
        ],
        scratch_shapes=[
            pltpu.VMEM((nslots, n_e), jnp.float32),
            pltpu.VMEM((nslots, n_e), jnp.float32),
        ],
    )
    pos2, pb, meta = routing(x_flat, gwt)
    posf = pos2.reshape(nslots)

    xs, ps = _sc_scatter(x_flat, posf, pb, p_rows)

    grid_spec = pltpu.PrefetchScalarGridSpec(
        num_scalar_prefetch=1,
        grid=(nsteps,),
        in_specs=[
            pl.BlockSpec((BN, d), lambda s_, m: (m[2, s_], 0)),
            pl.BlockSpec((1, HT, d), lambda s_, m: (m[0, s_], m[1, s_], 0)),
            pl.BlockSpec((1, HT, d), lambda s_, m: (m[0, s_], m[1, s_], 0)),
            pl.BlockSpec((1, d, HT), lambda s_, m: (m[0, s_], 0, m[1, s_])),
            pl.BlockSpec((BN, PW), lambda s_, m: (m[2, s_], 0)),
        ],
        out_specs=pl.BlockSpec((BN, d), lambda s_, m: (m[6, s_], 0)),
        scratch_shapes=[pltpu.VMEM((n, d), jnp.float32)],
    )
    ys = pl.pallas_call(
        functools.partial(_ffn_body, BN),
        grid_spec=grid_spec,
        out_shape=jax.ShapeDtypeStruct((p_rows, d), jnp.float32),
    )(meta, xs, w1, w2, w3, ps)

    out = _sc_combine(ys, posf, n, d)
    return out.reshape(b, s, d)


# speedup vs baseline: 1.7748x; 1.0556x over previous
import functools

import jax
import jax.numpy as jnp
from jax import lax
from jax.experimental import pallas as pl
from jax.experimental.pallas import tpu as pltpu
from jax.experimental.pallas import tpu_sc as plsc

BN = 512
HT = 1024
W = 16
PW = 128


def _routing_body(nslots, n_e, bn, nsteps, nh, x_ref, gwt_ref, pos_ref, pb_ref,
                  meta_ref, oh_ref, ex_ref):
    n = x_ref.shape[0]
    scores = jnp.dot(x_ref[...], gwt_ref[...],
                     preferred_element_type=jnp.float32)
    cols = lax.broadcasted_iota(jnp.int32, (n, n_e), 1)
    s0 = jnp.max(scores, axis=1, keepdims=True)
    e0 = jnp.min(jnp.where(scores == s0, cols, n_e), axis=1, keepdims=True)
    masked = jnp.where(cols == e0, -jnp.inf, scores)
    s1 = jnp.max(masked, axis=1, keepdims=True)
    e1 = jnp.min(jnp.where(masked == s1, cols, n_e), axis=1, keepdims=True)
    t = jnp.exp(s1 - s0)
    denom = 1.0 + t
    p0 = 1.0 / denom
    p1 = t / denom

    oh_ref[:n, :] = (cols == e0).astype(jnp.float32)
    oh_ref[n:, :] = (cols == e1).astype(jnp.float32)

    ch = 512
    tri = (lax.broadcasted_iota(jnp.int32, (ch, ch), 0)
           > lax.broadcasted_iota(jnp.int32, (ch, ch), 1)).astype(jnp.float32)

    def chunk(i, carry):
        blk = oh_ref[pl.ds(i * ch, ch), :]
        ex_ref[pl.ds(i * ch, ch), :] = (
            jnp.dot(tri, blk, preferred_element_type=jnp.float32) + carry)
        return carry + jnp.sum(blk, axis=0, keepdims=True)

    counts = lax.fori_loop(0, (2 * n) // ch, chunk,
                           jnp.zeros((1, n_e), jnp.float32))
    ranks = jnp.sum(ex_ref[...] * oh_ref[...], axis=1, keepdims=True)

    ci = counts.astype(jnp.int32)
    nb = (ci + (bn - 1)) // bn
    pc = nb * bn
    upper = (lax.broadcasted_iota(jnp.int32, (n_e, n_e), 0)
             < lax.broadcasted_iota(jnp.int32, (n_e, n_e), 1)).astype(jnp.float32)
    aoff = jnp.dot(pc.astype(jnp.float32), upper,
                   preferred_element_type=jnp.float32)
    asel = jnp.sum(oh_ref[...] * aoff, axis=1, keepdims=True)
    pos_ref[...] = (asel + ranks).astype(jnp.int32)
    pb_ref[...] = jnp.broadcast_to(jnp.concatenate([p0, p1], axis=0),
                                   (nslots, pb_ref.shape[1]))

    steps_e = nb * nh
    cums = jnp.dot(steps_e.astype(jnp.float32), upper,
                   preferred_element_type=jnp.float32).astype(jnp.int32)
    cums_inc = cums + steps_e
    nreal = jnp.sum(steps_e, axis=1, keepdims=True)
    sidx = lax.broadcasted_iota(jnp.int32, (1, nsteps), 1)
    ge = (sidx >= cums_inc.reshape(n_e, 1)).astype(jnp.int32)
    e_of_s = jnp.sum(ge, axis=0, keepdims=True)
    e_clamped = jnp.minimum(e_of_s, n_e - 1)
    sel = (e_clamped == lax.broadcasted_iota(jnp.int32, (n_e, nsteps), 0)
           ).astype(jnp.int32)
    base_sel = jnp.sum(sel * cums.reshape(n_e, 1), axis=0, keepdims=True)
    nb_sel = jnp.sum(sel * nb.reshape(n_e, 1), axis=0, keepdims=True)
    ab_sel = jnp.sum(sel * (aoff.astype(jnp.int32) // bn).reshape(n_e, 1),
                     axis=0, keepdims=True)
    r = sidx - base_sel
    nb_safe = jnp.maximum(nb_sel, 1)
    h = (r.astype(jnp.float32) / nb_safe.astype(jnp.float32)).astype(jnp.int32)
    j = r - h * nb_sel
    valid = (sidx < nreal).astype(jnp.int32)
    dump = (nslots + n_e * bn) // bn - 1
    rb = jnp.where(valid == 1, ab_sel + j, dump)
    h = jnp.where(valid == 1, h, nh - 1)
    j = jnp.where(valid == 1, j, 0)
    islast = jnp.where(valid == 1, (h == nh - 1).astype(jnp.int32), 0)
    meta_ref[0:1, :] = e_clamped
    meta_ref[1:2, :] = h
    meta_ref[2:3, :] = rb
    meta_ref[3:4, :] = j
    meta_ref[4:5, :] = islast
    meta_ref[5:6, :] = valid
    meta_ref[6:7, :] = jnp.where(islast == 1, rb, dump)
    meta_ref[7:8, :] = jnp.zeros((1, nsteps), jnp.int32)


def _ffn_body(bn, meta_ref, xs_ref, w1_ref, w2_ref, w3_ref, ps_ref, ys_ref,
              acc_ref):
    s = pl.program_id(0)
    h = meta_ref[1, s]
    j = meta_ref[3, s]
    islast = meta_ref[4, s]
    valid = meta_ref[5, s]
    base = j * bn

    @pl.when(valid == 1)
    def _():
        xb = xs_ref[...].astype(jnp.bfloat16)
        w1b = w1_ref[0].astype(jnp.bfloat16)
        w2b = w2_ref[0].astype(jnp.bfloat16)
        w3b = w3_ref[0].astype(jnp.bfloat16)
        dn = (((1,), (1,)), ((), ()))
        a = lax.dot_general(xb, w1b, dn, preferred_element_type=jnp.float32)
        b = lax.dot_general(xb, w2b, dn, preferred_element_type=jnp.float32)
        hid = (a * (1.0 / (1.0 + jnp.exp(-a)))) * b
        part = lax.dot_general(hid.astype(jnp.bfloat16), w3b, dn,
                               preferred_element_type=jnp.float32)
        prev = acc_ref[pl.ds(base, bn), :]
        acc = jnp.where(h == 0, part, prev + part)
        acc_ref[pl.ds(base, bn), :] = acc

        @pl.when(islast == 1)
        def _():
            ys_ref[...] = acc * ps_ref[:, 0:1]


def _sc_scatter(x_flat, posf, pb, p_rows):
    n, d = x_flat.shape
    mesh = plsc.VectorSubcoreMesh(core_axis_name="c", subcore_axis_name="s")
    nworkers = 32
    tpw = n // nworkers
    nchunks = tpw // W

    @functools.partial(
        pl.kernel,
        out_type=[
            jax.ShapeDtypeStruct((p_rows, d), jnp.float32),
            jax.ShapeDtypeStruct((p_rows, PW), jnp.float32),
        ],
        mesh=mesh,
        scratch_types=[
            pltpu.VMEM((W, d), jnp.float32),
            pltpu.VMEM((W, d), jnp.float32),
            pltpu.VMEM((W, PW), jnp.float32),
            pltpu.VMEM((W, PW), jnp.float32),
            pltpu.VMEM((W, PW), jnp.float32),
            pltpu.VMEM((W, PW), jnp.float32),
            pltpu.VMEM((W,), jnp.int32),
            pltpu.VMEM((W,), jnp.int32),
            pltpu.VMEM((W,), jnp.int32),
            pltpu.VMEM((W,), jnp.int32),
            pltpu.SemaphoreType.DMA,
            pltpu.SemaphoreType.DMA,
            pltpu.SemaphoreType.DMA,
            pltpu.SemaphoreType.DMA,
        ],
    )
    def k(x_hbm, pos_hbm, pb_hbm, xs_hbm, ps_hbm, xva, xvb,
          pv0a, pv0b, pv1a, pv1b, i0a, i0b, i1a, i1b, sla, slb, ssa, ssb):
        wid = lax.axis_index("s") * 2 + lax.axis_index("c")
        xv = (xva, xvb)
        pv0 = (pv0a, pv0b)
        pv1 = (pv1a, pv1b)
        i0 = (i0a, i0b)
        i1 = (i1a, i1b)
        sl = (sla, slb)
        ss = (ssa, ssb)

        def start(cc):
            bsl = cc % 2
            base = wid * tpw + cc * W
            pltpu.sync_copy(pos_hbm.at[pl.ds(base, W)], i0[bsl])
            pltpu.sync_copy(pos_hbm.at[pl.ds(n + base, W)], i1[bsl])
            pltpu.sync_copy(pb_hbm.at[pl.ds(base, W)], pv0[bsl])
            pltpu.sync_copy(pb_hbm.at[pl.ds(n + base, W)], pv1[bsl])
            return pltpu.async_copy(x_hbm.at[pl.ds(base, W)], xv[bsl], sl[bsl])

        loads = start(0)
        stores = [None, None]
        for cc in range(nchunks):
            bsl = cc % 2
            loads.wait()
            cps = (
                pltpu.async_copy(xv[bsl], xs_hbm.at[i0[bsl]], ss[bsl]),
                pltpu.async_copy(xv[bsl], xs_hbm.at[i1[bsl]], ss[bsl]),
                pltpu.async_copy(pv0[bsl], ps_hbm.at[i0[bsl]], ss[bsl]),
                pltpu.async_copy(pv1[bsl], ps_hbm.at[i1[bsl]], ss[bsl]),
            )
            stores[bsl] = cps
            if cc + 1 < nchunks:
                nxt = (cc + 1) % 2
                if stores[nxt] is not None:
                    for cp in stores[nxt]:
                        cp.wait()
                    stores[nxt] = None
                loads = start(cc + 1)
        for group in stores:
            if group is not None:
                for cp in group:
                    cp.wait()

    return k(x_flat, posf, pb)


def _sc_combine(ys, posf, n, d):
    mesh = plsc.VectorSubcoreMesh(core_axis_name="c", subcore_axis_name="s")
    nworkers = 32
    tpw = n // nworkers
    nchunks = tpw // W

    @functools.partial(
        pl.kernel,
        out_type=jax.ShapeDtypeStruct((n, d), jnp.float32),
        mesh=mesh,
        scratch_types=[
            pltpu.VMEM((W, d), jnp.float32),
            pltpu.VMEM((W, d), jnp.float32),
            pltpu.VMEM((tpw,), jnp.int32),
            pltpu.VMEM((tpw,), jnp.int32),
            pltpu.SemaphoreType.DMA,
            pltpu.SemaphoreType.DMA,
            pltpu.SemaphoreType.DMA,
            pltpu.SemaphoreType.DMA,
            pltpu.SemaphoreType.DMA,
            pltpu.SemaphoreType.DMA,
        ],
    )
    def k(ys_hbm, pos_hbm, out_hbm, g0a, g0b,
          i0all, i1all, s0a, s0b, s1a, s1b, soa, sob):
        wid = lax.axis_index("s") * 2 + lax.axis_index("c")
        base0 = wid * tpw
        pltpu.sync_copy(pos_hbm.at[pl.ds(base0, tpw)], i0all)
        pltpu.sync_copy(pos_hbm.at[pl.ds(n + base0, tpw)], i1all)
        g0 = (g0a, g0b)
        s0 = (s0a, s0b)
        s1 = (s1a, s1b)
        so = (soa, sob)

        def start0(cc):
            bsl = cc % 2
            return pltpu.async_copy(
                ys_hbm.at[i0all.at[pl.ds(cc * W, W)]], g0[bsl], s0[bsl])

        def start1(cc):
            bsl = cc % 2
            return pltpu.async_copy(
                ys_hbm.at[i1all.at[pl.ds(cc * W, W)]], g0[bsl], s1[bsl],
                add=True)

        load0 = start0(0)
        stores = [None, None]
        for cc in range(nchunks):
            bsl = cc % 2
            load0.wait()
            adds = start1(cc)
            if cc + 1 < nchunks:
                nxt = (cc + 1) % 2
                if stores[nxt] is not None:
                    stores[nxt].wait()
                    stores[nxt] = None
                load0 = start0(cc + 1)
            adds.wait()
            stores[bsl] = pltpu.async_copy(
                g0[bsl], out_hbm.at[pl.ds(base0 + cc * W, W)], so[bsl])
        for st in stores:
            if st is not None:
                st.wait()

    return k(ys, posf)


def kernel(x, gate_w, w1, w2, w3):
    b, s, d = x.shape
    n_e, _ = gate_w.shape
    hdim = w1.shape[1]
    n = b * s
    nslots = 2 * n
    nh = hdim // HT
    p_rows = nslots + n_e * BN
    nsteps = (p_rows // BN) * nh

    x_flat = x.reshape(n, d)
    gwt = gate_w.T

    routing = pl.pallas_call(
        functools.partial(_routing_body, nslots, n_e, BN, nsteps, nh),
        out_shape=[
            jax.ShapeDtypeStruct((nslots, 1), jnp.int32),
            jax.ShapeDtypeStruct((nslots, PW), jnp.float32),
            jax.ShapeDtypeStruct((8, nsteps), jnp.int32),
        ],
        scratch_shapes=[
            pltpu.VMEM((nslots, n_e), jnp.float32),
            pltpu.VMEM((nslots, n_e), jnp.float32),
        ],
    )
    pos2, pb, meta = routing(x_flat, gwt)
    posf = pos2.reshape(nslots)

    xs, ps = _sc_scatter(x_flat, posf, pb, p_rows)

    grid_spec = pltpu.PrefetchScalarGridSpec(
        num_scalar_prefetch=1,
        grid=(nsteps,),
        in_specs=[
            pl.BlockSpec((BN, d), lambda s_, m: (m[2, s_], 0)),
            pl.BlockSpec((1, HT, d), lambda s_, m: (m[0, s_], m[1, s_], 0)),
            pl.BlockSpec((1, HT, d), lambda s_, m: (m[0, s_], m[1, s_], 0)),
            pl.BlockSpec((1, d, HT), lambda s_, m: (m[0, s_], 0, m[1, s_])),
            pl.BlockSpec((BN, PW), lambda s_, m: (m[2, s_], 0)),
        ],
        out_specs=pl.BlockSpec((BN, d), lambda s_, m: (m[6, s_], 0)),
        scratch_shapes=[pltpu.VMEM((n, d), jnp.float32)],
    )
    ys = pl.pallas_call(
        functools.partial(_ffn_body, BN),
        grid_spec=grid_spec,
        out_shape=jax.ShapeDtypeStruct((p_rows, d), jnp.float32),
    )(meta, xs, w1, w2, w3, ps)

    out = _sc_combine(ys, posf, n, d)
    return out.reshape(b, s, d)
